# Initial kernel scaffold; baseline (speedup 1.0000x reference)
import jax
import jax.numpy as jnp
from jax.experimental import pallas as pl


def kernel(x, edge_index, W_gat, att_src, att_dst, bias, W_shared, b_shared):
    raise NotImplementedError("write your pallas kernel here")


# speedup vs baseline: 1.7944x; 1.7944x over previous
import functools
import jax
import jax.numpy as jnp
from jax.experimental import pallas as pl

_EB1 = 256
_EB2 = 128
_NB = 256


def _rup(a, b):
    return (a + b - 1) // b * b


def _a_kernel(x_ref, wg_ref, ws_ref, as_ref, b2_ref, hb_ref, a2_ref, s2_ref):
    x = x_ref[...]
    h = jnp.dot(x, wg_ref[...], preferred_element_type=jnp.float32)
    hb_ref[...] = h.astype(jnp.bfloat16)
    a2_ref[...] = jnp.dot(h, as_ref[...], preferred_element_type=jnp.float32)
    s2_ref[...] = (jnp.dot(x, ws_ref[...], preferred_element_type=jnp.float32)
                   + b2_ref[0:1, :])


def _b1_kernel(src_ref, dst_ref, a2_ref, ex_ref, den_ref, *, np_, eb, heads):
    iota = jax.lax.broadcasted_iota(jnp.int32, (eb, np_), 1)
    oh_s = jnp.where(iota == src_ref[...], 1.0, 0.0).astype(jnp.float32)
    oh_d = jnp.where(iota == dst_ref[...], 1.0, 0.0).astype(jnp.float32)
    a2 = a2_ref[...]
    g = (jnp.dot(oh_s, a2[:, 0:heads], preferred_element_type=jnp.float32)
         + jnp.dot(oh_d, a2[:, heads:2 * heads], preferred_element_type=jnp.float32))
    alpha = jnp.where(g > 0, g, 0.2 * g)
    ex = jnp.exp(alpha)
    ex_ref[...] = ex

    @pl.when(pl.program_id(0) == 0)
    def _():
        den_ref[...] = jnp.zeros_like(den_ref)

    den_ref[...] += jax.lax.dot_general(
        oh_d, ex, (((0,), (0,)), ((), ())),
        preferred_element_type=jnp.float32)


def _b2_kernel(src_ref, dst_ref, ex_ref, den_ref, hb_ref, out_ref, *, np_, eb, heads, ch):
    iota = jax.lax.broadcasted_iota(jnp.int32, (eb, np_), 1)
    oh_d32 = jnp.where(iota == dst_ref[...], 1.0, 0.0).astype(jnp.float32)
    oh_s = jnp.where(iota == src_ref[...], 1.0, 0.0).astype(jnp.bfloat16)
    den_g = jnp.dot(oh_d32, den_ref[...], preferred_element_type=jnp.float32)
    coef = ex_ref[...] / (den_g + 1e-16)
    hs = jnp.dot(oh_s, hb_ref[...], preferred_element_type=jnp.float32)
    parts = [hs[:, i * ch:(i + 1) * ch] * coef[:, i:i + 1]
             for i in range(heads)]
    msg = jnp.concatenate(parts, axis=1).astype(jnp.bfloat16)

    @pl.when(pl.program_id(0) == 0)
    def _():
        out_ref[...] = jnp.zeros_like(out_ref)

    out_ref[...] += jax.lax.dot_general(
        oh_d32.astype(jnp.bfloat16), msg, (((0,), (0,)), ((), ())),
        preferred_element_type=jnp.float32)


def _c_kernel(g_ref, s_ref, o_ref):
    o_ref[...] = g_ref[...] + s_ref[...]


def kernel(x, edge_index, W_gat, att_src, att_dst, bias, W_shared, b_shared):
    n, d_in = x.shape
    hc = W_gat.shape[1]
    heads = att_src.shape[1]
    ch = att_src.shape[2]
    np_ = _rup(n, _NB)

    eye = jnp.eye(heads, dtype=jnp.float32)
    a_s = jnp.einsum("hc,hj->hcj", att_src.reshape(heads, ch), eye).reshape(hc, heads)
    a_d = jnp.einsum("hc,hj->hcj", att_dst.reshape(heads, ch), eye).reshape(hc, heads)
    as_cat = jnp.concatenate([a_s, a_d], axis=1)
    b2 = jnp.broadcast_to((bias + b_shared)[None, :], (8, hc))

    xp = jnp.zeros((np_, d_in), jnp.float32).at[:n].set(x)

    loops = jnp.arange(n, dtype=jnp.int32)
    src = jnp.concatenate([edge_index[0].astype(jnp.int32), loops])
    dst = jnp.concatenate([edge_index[1].astype(jnp.int32), loops])
    e2 = _rup(src.shape[0], max(_EB1, _EB2))
    sent = jnp.int32(np_)
    srcp = jnp.full((e2, 1), sent, jnp.int32).at[:src.shape[0], 0].set(src)
    dstp = jnp.full((e2, 1), sent, jnp.int32).at[:dst.shape[0], 0].set(dst)

    hb, a2, s2 = pl.pallas_call(
        _a_kernel,
        grid=(np_ // _NB,),
        in_specs=[
            pl.BlockSpec((_NB, d_in), lambda i: (i, 0)),
            pl.BlockSpec((d_in, hc), lambda i: (0, 0)),
            pl.BlockSpec((d_in, hc), lambda i: (0, 0)),
            pl.BlockSpec((hc, 2 * heads), lambda i: (0, 0)),
            pl.BlockSpec((8, hc), lambda i: (0, 0)),
        ],
        out_specs=[
            pl.BlockSpec((_NB, hc), lambda i: (i, 0)),
            pl.BlockSpec((_NB, 2 * heads), lambda i: (i, 0)),
            pl.BlockSpec((_NB, hc), lambda i: (i, 0)),
        ],
        out_shape=[
            jax.ShapeDtypeStruct((np_, hc), jnp.bfloat16),
            jax.ShapeDtypeStruct((np_, 2 * heads), jnp.float32),
            jax.ShapeDtypeStruct((np_, hc), jnp.float32),
        ],
    )(xp, W_gat, W_shared, as_cat, b2)

    ex, den = pl.pallas_call(
        functools.partial(_b1_kernel, np_=np_, eb=_EB1, heads=heads),
        grid=(e2 // _EB1,),
        in_specs=[
            pl.BlockSpec((_EB1, 1), lambda i: (i, 0)),
            pl.BlockSpec((_EB1, 1), lambda i: (i, 0)),
            pl.BlockSpec((np_, 2 * heads), lambda i: (0, 0)),
        ],
        out_specs=[
            pl.BlockSpec((_EB1, heads), lambda i: (i, 0)),
            pl.BlockSpec((np_, heads), lambda i: (0, 0)),
        ],
        out_shape=[
            jax.ShapeDtypeStruct((e2, heads), jnp.float32),
            jax.ShapeDtypeStruct((np_, heads), jnp.float32),
        ],
    )(srcp, dstp, a2)

    gat = pl.pallas_call(
        functools.partial(_b2_kernel, np_=np_, eb=_EB2, heads=heads, ch=ch),
        grid=(e2 // _EB2,),
        in_specs=[
            pl.BlockSpec((_EB2, 1), lambda i: (i, 0)),
            pl.BlockSpec((_EB2, 1), lambda i: (i, 0)),
            pl.BlockSpec((_EB2, heads), lambda i: (i, 0)),
            pl.BlockSpec((np_, heads), lambda i: (0, 0)),
            pl.BlockSpec((np_, hc), lambda i: (0, 0)),
        ],
        out_specs=pl.BlockSpec((np_, hc), lambda i: (0, 0)),
        out_shape=jax.ShapeDtypeStruct((np_, hc), jnp.float32),
    )(srcp, dstp, ex, den, hb)

    out = pl.pallas_call(
        _c_kernel,
        grid=(np_ // _NB,),
        in_specs=[
            pl.BlockSpec((_NB, hc), lambda i: (i, 0)),
            pl.BlockSpec((_NB, hc), lambda i: (i, 0)),
        ],
        out_specs=pl.BlockSpec((_NB, hc), lambda i: (i, 0)),
        out_shape=jax.ShapeDtypeStruct((np_, hc), jnp.float32),
    )(gat, s2)
    return out[:n]


# speedup vs baseline: 1.8903x; 1.0534x over previous
import functools
import jax
import jax.numpy as jnp
from jax.experimental import pallas as pl

_EB1 = 512
_EB2 = 256
_NB = 256


def _rup(a, b):
    return (a + b - 1) // b * b


def _a_kernel(x_ref, wg_ref, ws_ref, as_ref, b2_ref, hb_ref, a2_ref, s2_ref):
    x = x_ref[...]
    h = jnp.dot(x, wg_ref[...], preferred_element_type=jnp.float32)
    hb_ref[...] = h.astype(jnp.bfloat16)
    a2_ref[...] = jnp.dot(h, as_ref[...], preferred_element_type=jnp.float32)
    s2_ref[...] = (jnp.dot(x, ws_ref[...], preferred_element_type=jnp.float32)
                   + b2_ref[0:1, :])


def _b1_kernel(src_ref, dst_ref, a2_ref, ex_ref, den_ref, *, np_, eb, heads):
    iota = jax.lax.broadcasted_iota(jnp.int32, (eb, np_), 1)
    oh_s = jnp.where(iota == src_ref[...], 1.0, 0.0).astype(jnp.float32)
    oh_d = jnp.where(iota == dst_ref[...], 1.0, 0.0).astype(jnp.float32)
    a2 = a2_ref[...]
    g = (jnp.dot(oh_s, a2[:, 0:heads], preferred_element_type=jnp.float32)
         + jnp.dot(oh_d, a2[:, heads:2 * heads], preferred_element_type=jnp.float32))
    alpha = jnp.where(g > 0, g, 0.2 * g)
    ex = jnp.exp(alpha)
    ex_ref[...] = ex

    @pl.when(pl.program_id(0) == 0)
    def _():
        den_ref[...] = jnp.zeros_like(den_ref)

    den_ref[...] += jax.lax.dot_general(
        oh_d, ex, (((0,), (0,)), ((), ())),
        preferred_element_type=jnp.float32)


def _b2_kernel(src_ref, dst_ref, ex_ref, den_ref, hb_ref, out_ref, *, np_, eb, heads, ch):
    iota = jax.lax.broadcasted_iota(jnp.int32, (eb, np_), 1)
    oh_d32 = jnp.where(iota == dst_ref[...], 1.0, 0.0).astype(jnp.float32)
    oh_s = jnp.where(iota == src_ref[...], 1.0, 0.0).astype(jnp.bfloat16)
    den_g = jnp.dot(oh_d32, den_ref[...], preferred_element_type=jnp.float32)
    coef = ex_ref[...] / (den_g + 1e-16)
    hs = jnp.dot(oh_s, hb_ref[...], preferred_element_type=jnp.float32)
    parts = [hs[:, i * ch:(i + 1) * ch] * coef[:, i:i + 1]
             for i in range(heads)]
    msg = jnp.concatenate(parts, axis=1).astype(jnp.bfloat16)

    @pl.when(pl.program_id(0) == 0)
    def _():
        out_ref[...] = jnp.zeros_like(out_ref)

    out_ref[...] += jax.lax.dot_general(
        oh_d32.astype(jnp.bfloat16), msg, (((0,), (0,)), ((), ())),
        preferred_element_type=jnp.float32)


def _c_kernel(g_ref, s_ref, o_ref):
    o_ref[...] = g_ref[...] + s_ref[...]


def kernel(x, edge_index, W_gat, att_src, att_dst, bias, W_shared, b_shared):
    n, d_in = x.shape
    hc = W_gat.shape[1]
    heads = att_src.shape[1]
    ch = att_src.shape[2]
    np_ = _rup(n, _NB)

    eye = jnp.eye(heads, dtype=jnp.float32)
    a_s = jnp.einsum("hc,hj->hcj", att_src.reshape(heads, ch), eye).reshape(hc, heads)
    a_d = jnp.einsum("hc,hj->hcj", att_dst.reshape(heads, ch), eye).reshape(hc, heads)
    as_cat = jnp.concatenate([a_s, a_d], axis=1)
    b2 = jnp.broadcast_to((bias + b_shared)[None, :], (8, hc))

    xp = jnp.zeros((np_, d_in), jnp.float32).at[:n].set(x)

    loops = jnp.arange(n, dtype=jnp.int32)
    src = jnp.concatenate([edge_index[0].astype(jnp.int32), loops])
    dst = jnp.concatenate([edge_index[1].astype(jnp.int32), loops])
    e2 = _rup(src.shape[0], max(_EB1, _EB2))
    sent = jnp.int32(np_)
    srcp = jnp.full((e2, 1), sent, jnp.int32).at[:src.shape[0], 0].set(src)
    dstp = jnp.full((e2, 1), sent, jnp.int32).at[:dst.shape[0], 0].set(dst)

    hb, a2, s2 = pl.pallas_call(
        _a_kernel,
        grid=(np_ // _NB,),
        in_specs=[
            pl.BlockSpec((_NB, d_in), lambda i: (i, 0)),
            pl.BlockSpec((d_in, hc), lambda i: (0, 0)),
            pl.BlockSpec((d_in, hc), lambda i: (0, 0)),
            pl.BlockSpec((hc, 2 * heads), lambda i: (0, 0)),
            pl.BlockSpec((8, hc), lambda i: (0, 0)),
        ],
        out_specs=[
            pl.BlockSpec((_NB, hc), lambda i: (i, 0)),
            pl.BlockSpec((_NB, 2 * heads), lambda i: (i, 0)),
            pl.BlockSpec((_NB, hc), lambda i: (i, 0)),
        ],
        out_shape=[
            jax.ShapeDtypeStruct((np_, hc), jnp.bfloat16),
            jax.ShapeDtypeStruct((np_, 2 * heads), jnp.float32),
            jax.ShapeDtypeStruct((np_, hc), jnp.float32),
        ],
    )(xp, W_gat, W_shared, as_cat, b2)

    ex, den = pl.pallas_call(
        functools.partial(_b1_kernel, np_=np_, eb=_EB1, heads=heads),
        grid=(e2 // _EB1,),
        in_specs=[
            pl.BlockSpec((_EB1, 1), lambda i: (i, 0)),
            pl.BlockSpec((_EB1, 1), lambda i: (i, 0)),
            pl.BlockSpec((np_, 2 * heads), lambda i: (0, 0)),
        ],
        out_specs=[
            pl.BlockSpec((_EB1, heads), lambda i: (i, 0)),
            pl.BlockSpec((np_, heads), lambda i: (0, 0)),
        ],
        out_shape=[
            jax.ShapeDtypeStruct((e2, heads), jnp.float32),
            jax.ShapeDtypeStruct((np_, heads), jnp.float32),
        ],
    )(srcp, dstp, a2)

    gat = pl.pallas_call(
        functools.partial(_b2_kernel, np_=np_, eb=_EB2, heads=heads, ch=ch),
        grid=(e2 // _EB2,),
        in_specs=[
            pl.BlockSpec((_EB2, 1), lambda i: (i, 0)),
            pl.BlockSpec((_EB2, 1), lambda i: (i, 0)),
            pl.BlockSpec((_EB2, heads), lambda i: (i, 0)),
            pl.BlockSpec((np_, heads), lambda i: (0, 0)),
            pl.BlockSpec((np_, hc), lambda i: (0, 0)),
        ],
        out_specs=pl.BlockSpec((np_, hc), lambda i: (0, 0)),
        out_shape=jax.ShapeDtypeStruct((np_, hc), jnp.float32),
    )(srcp, dstp, ex, den, hb)

    out = pl.pallas_call(
        _c_kernel,
        grid=(np_ // _NB,),
        in_specs=[
            pl.BlockSpec((_NB, hc), lambda i: (i, 0)),
            pl.BlockSpec((_NB, hc), lambda i: (i, 0)),
        ],
        out_specs=pl.BlockSpec((_NB, hc), lambda i: (i, 0)),
        out_shape=jax.ShapeDtypeStruct((np_, hc), jnp.float32),
    )(gat, s2)
    return out[:n]
